# baseline (speedup 1.0000x reference)
import functools

import jax
import jax.numpy as jnp
from jax import lax
from jax.experimental import pallas as pl
from jax.experimental.pallas import tpu as pltpu
from jax.experimental.pallas import tpu_sc as plsc

EMBED = 64
BATCH = 16384
LANES = 16
PAIR = 2 * EMBED
NUM_CORES = 2
NUM_SUBCORES = 16
NW = NUM_CORES * NUM_SUBCORES
CHUNK = 128
NCH = BATCH // (NW * CHUNK)

_MESH = plsc.VectorSubcoreMesh(
    core_axis_name="c", subcore_axis_name="s",
    num_cores=NUM_CORES, num_subcores=NUM_SUBCORES)


@functools.partial(
    pl.kernel,
    out_type=jax.ShapeDtypeStruct((NW, NCH, CHUNK, EMBED), jnp.float32),
    mesh=_MESH,
    scratch_types=[
        pltpu.VMEM((NCH, CHUNK), jnp.int32),
        pltpu.VMEM((CHUNK,), jnp.int32),
        pltpu.VMEM((CHUNK, PAIR), jnp.float32),
        pltpu.VMEM((CHUNK, EMBED), jnp.float32),
        pltpu.VMEM((CHUNK, EMBED), jnp.float32),
        pltpu.SemaphoreType.DMA,
        pltpu.SemaphoreType.DMA,
    ],
)
def _sc_embed(x_hbm, idx_hbm, table_hbm, out_hbm,
              idx_v, pid_v, gath_v, x_v, out_v, gsem, xsem):
    wid = lax.axis_index("s") * NUM_CORES + lax.axis_index("c")

    pltpu.sync_copy(idx_hbm.at[wid], idx_v)

    def chunk_body(c, carry):
        for g in range(CHUNK // LANES):
            sl = pl.ds(g * LANES, LANES)
            pid_v[sl] = lax.shift_right_logical(idx_v[c, sl], 1)
        gath_cp = pltpu.async_copy(table_hbm.at[pid_v], gath_v, gsem)
        x_cp = pltpu.async_copy(x_hbm.at[wid].at[c], x_v, xsem)
        gath_cp.wait()
        x_cp.wait()

        for g in range(CHUNK // LANES):
            svec = lax.bitwise_and(idx_v[c, pl.ds(g * LANES, LANES)], 1)
            hvec = svec * EMBED
            for l in range(LANES):
                h = hvec[l]
                j = g * LANES + l
                for d in range(EMBED // LANES):
                    dsl = pl.ds(d * LANES, LANES)
                    out_v[j, dsl] = (
                        gath_v[j, pl.ds(h + d * LANES, LANES)]
                        * x_v[j, dsl])

        pltpu.sync_copy(out_v, out_hbm.at[wid].at[c])
        return carry

    lax.fori_loop(0, NCH, chunk_body, 0)


def kernel(x, id, table):
    idx = id.astype(jnp.int32).reshape(NW, NCH, CHUNK)
    x_r = x.reshape(NW, NCH, CHUNK, EMBED)
    table_p = table.reshape(table.shape[0] // 2, PAIR)
    out = _sc_embed(x_r, idx, table_p)
    return out.reshape(BATCH, EMBED)


# speedup vs baseline: 1.7112x; 1.7112x over previous
import functools

import jax
import jax.numpy as jnp
from jax import lax
from jax.experimental import pallas as pl
from jax.experimental.pallas import tpu as pltpu
from jax.experimental.pallas import tpu_sc as plsc

EMBED = 64
BATCH = 16384
LANES = 16
NUM_CORES = 2
NUM_SUBCORES = 16
NW = NUM_CORES * NUM_SUBCORES
CHUNK = 128
NCH = BATCH // (NW * CHUNK)
NBUF = 2

_MESH = plsc.VectorSubcoreMesh(
    core_axis_name="c", subcore_axis_name="s",
    num_cores=NUM_CORES, num_subcores=NUM_SUBCORES)


@functools.partial(
    pl.kernel,
    out_type=jax.ShapeDtypeStruct((NW, NCH, CHUNK, EMBED), jnp.float32),
    mesh=_MESH,
    scratch_types=[
        pltpu.VMEM((NCH, CHUNK), jnp.int32),
        pltpu.VMEM((NBUF, CHUNK, EMBED), jnp.float32),
        pltpu.VMEM((NBUF, CHUNK, EMBED), jnp.float32),
        pltpu.VMEM((NBUF, CHUNK, EMBED), jnp.float32),
        [pltpu.SemaphoreType.DMA] * NBUF,
        [pltpu.SemaphoreType.DMA] * NBUF,
        [pltpu.SemaphoreType.DMA] * NBUF,
    ],
)
def _sc_embed(x_hbm, idx_hbm, table_hbm, out_hbm,
              idx_v, rows_v, x_v, out_v, gsems, xsems, osems):
    wid = lax.axis_index("s") * NUM_CORES + lax.axis_index("c")

    pltpu.sync_copy(idx_hbm.at[wid], idx_v)

    def issue_chunk(c, b):
        pltpu.async_copy(x_hbm.at[wid].at[c], x_v.at[b], xsems[b])

        def issue(g, carry):
            vec = idx_v[c, pl.ds(g * LANES, LANES)]
            for l in range(LANES):
                pltpu.async_copy(
                    table_hbm.at[vec[l]],
                    rows_v.at[b].at[g * LANES + l],
                    gsems[b])
            return carry

        lax.fori_loop(0, CHUNK // LANES, issue, 0)

    def drain_chunk(c, b):
        pltpu.make_async_copy(
            table_hbm.at[pl.ds(0, CHUNK)], rows_v.at[b], gsems[b]).wait()
        pltpu.make_async_copy(
            x_hbm.at[wid].at[c], x_v.at[b], xsems[b]).wait()

    issue_chunk(0, 0)
    for c in range(NCH):
        b = c % NBUF
        if c + 1 < NCH:
            issue_chunk(c + 1, (c + 1) % NBUF)
        drain_chunk(c, b)
        if c >= NBUF:
            pltpu.make_async_copy(
                out_v.at[b], out_hbm.at[wid].at[c - NBUF], osems[b]).wait()

        def mul(r, carry):
            for d in range(EMBED // LANES):
                sl = pl.ds(d * LANES, LANES)
                out_v[b, r, sl] = rows_v[b, r, sl] * x_v[b, r, sl]
            return carry

        lax.fori_loop(0, CHUNK, mul, 0)
        pltpu.async_copy(out_v.at[b], out_hbm.at[wid].at[c], osems[b])

    for c in range(NCH - NBUF, NCH):
        b = c % NBUF
        pltpu.make_async_copy(
            out_v.at[b], out_hbm.at[wid].at[c], osems[b]).wait()


def kernel(x, id, table):
    idx = id.astype(jnp.int32).reshape(NW, NCH, CHUNK)
    x_r = x.reshape(NW, NCH, CHUNK, EMBED)
    out = _sc_embed(x_r, idx, table)
    return out.reshape(BATCH, EMBED)


# speedup vs baseline: 2.3349x; 1.3645x over previous
import functools

import jax
import jax.numpy as jnp
from jax import lax
from jax.experimental import pallas as pl
from jax.experimental.pallas import tpu as pltpu
from jax.experimental.pallas import tpu_sc as plsc

EMBED = 64
BATCH = 16384
LANES = 16
TILE_R = 8
NUM_CORES = 2
NUM_SUBCORES = 16
NW = NUM_CORES * NUM_SUBCORES
CHUNK = 32
NCH = BATCH // (NW * CHUNK)

_MESH = plsc.VectorSubcoreMesh(
    core_axis_name="c", subcore_axis_name="s",
    num_cores=NUM_CORES, num_subcores=NUM_SUBCORES)


@functools.partial(
    pl.kernel,
    out_type=jax.ShapeDtypeStruct((NW, NCH, CHUNK, EMBED), jnp.float32),
    mesh=_MESH,
    scratch_types=[
        pltpu.VMEM((NCH, CHUNK), jnp.int32),
        pltpu.VMEM((2, CHUNK, TILE_R, EMBED), jnp.float32),
        pltpu.VMEM((2, CHUNK, EMBED), jnp.float32),
        pltpu.VMEM((2, CHUNK, EMBED), jnp.float32),
        [pltpu.SemaphoreType.DMA] * 2,
        [pltpu.SemaphoreType.DMA] * 2,
        [pltpu.SemaphoreType.DMA] * 2,
    ],
)
def _sc_embed(x_hbm, idx_hbm, table_hbm, out_hbm,
              idx_v, gath_v, x_v, out_v, gsems, xsems, osems):
    wid = lax.axis_index("s") * NUM_CORES + lax.axis_index("c")

    pltpu.sync_copy(idx_hbm.at[wid], idx_v)

    def issue_chunk(c, b):
        pltpu.async_copy(x_hbm.at[wid].at[c], x_v.at[b], xsems[b])
        for g in range(CHUNK // LANES):
            tvec = lax.shift_right_logical(
                idx_v[c, pl.ds(g * LANES, LANES)], 3)
            for l in range(LANES):
                pltpu.async_copy(
                    table_hbm.at[tvec[l]],
                    gath_v.at[b].at[g * LANES + l],
                    gsems[b])

    def process_chunk(c, b):
        pltpu.make_async_copy(
            table_hbm.at[pl.ds(0, CHUNK)], gath_v.at[b], gsems[b]).wait()
        pltpu.make_async_copy(
            x_hbm.at[wid].at[0], x_v.at[b], xsems[b]).wait()
        @pl.when(c >= 2)
        def _():
            pltpu.make_async_copy(
                out_v.at[b], out_hbm.at[wid].at[0], osems[b]).wait()

        for g in range(CHUNK // LANES):
            svec = lax.bitwise_and(idx_v[c, pl.ds(g * LANES, LANES)], 7)
            for l in range(LANES):
                s = svec[l]
                j = g * LANES + l
                for d in range(EMBED // LANES):
                    dsl = pl.ds(d * LANES, LANES)
                    out_v[b, j, dsl] = gath_v[b, j, s, dsl] * x_v[b, j, dsl]

        pltpu.async_copy(out_v.at[b], out_hbm.at[wid].at[c], osems[b])

    issue_chunk(0, 0)

    def pair_body(i, carry):
        c0 = i * 2
        issue_chunk(c0 + 1, 1)
        process_chunk(c0, 0)

        @pl.when(c0 + 2 < NCH)
        def _():
            issue_chunk(c0 + 2, 0)

        process_chunk(c0 + 1, 1)
        return carry

    lax.fori_loop(0, NCH // 2, pair_body, 0)

    for b in range(2):
        pltpu.make_async_copy(
            out_v.at[b], out_hbm.at[wid].at[0], osems[b]).wait()


def kernel(x, id, table):
    idx = id.astype(jnp.int32).reshape(NW, NCH, CHUNK)
    x_r = x.reshape(NW, NCH, CHUNK, EMBED)
    table_t = table.reshape(table.shape[0] // TILE_R, TILE_R, EMBED)
    out = _sc_embed(x_r, idx, table_t)
    return out.reshape(BATCH, EMBED)


# speedup vs baseline: 2.3455x; 1.0045x over previous
import functools

import jax
import jax.numpy as jnp
from jax import lax
from jax.experimental import pallas as pl
from jax.experimental.pallas import tpu as pltpu
from jax.experimental.pallas import tpu_sc as plsc

EMBED = 64
BATCH = 16384
LANES = 16
TILE_R = 8
NUM_CORES = 2
NUM_SUBCORES = 16
NW = NUM_CORES * NUM_SUBCORES
CHUNK = 32
HCH = CHUNK // 2
NCH = BATCH // (NW * CHUNK)

_MESH = plsc.VectorSubcoreMesh(
    core_axis_name="c", subcore_axis_name="s",
    num_cores=NUM_CORES, num_subcores=NUM_SUBCORES)


@functools.partial(
    pl.kernel,
    out_type=jax.ShapeDtypeStruct((NW, NCH, CHUNK, EMBED), jnp.float32),
    mesh=_MESH,
    scratch_types=[
        pltpu.VMEM((NCH, CHUNK), jnp.int32),
        pltpu.VMEM((2, HCH, TILE_R, EMBED), jnp.float32),
        pltpu.VMEM((2, HCH, TILE_R, EMBED), jnp.float32),
        pltpu.VMEM_SHARED((NUM_SUBCORES, 2, HCH, TILE_R, EMBED),
                          jnp.float32),
        pltpu.VMEM((2, CHUNK, EMBED), jnp.float32),
        pltpu.VMEM((2, CHUNK, EMBED), jnp.float32),
        [pltpu.SemaphoreType.DMA] * 2,
        [pltpu.SemaphoreType.DMA] * 2,
        [pltpu.SemaphoreType.DMA] * 2,
        [pltpu.SemaphoreType.DMA] * 2,
    ],
)
def _sc_embed(x_hbm, idx_hbm, table_hbm, out_hbm,
              idx_v, gath_v, gath2_v, spm, x_v, out_v,
              gsems, ssems, xsems, osems):
    cid = lax.axis_index("c")
    sid = lax.axis_index("s")
    wid = sid * NUM_CORES + cid

    pltpu.sync_copy(idx_hbm.at[wid], idx_v)

    def issue_chunk(c, b):
        pltpu.async_copy(x_hbm.at[wid].at[c], x_v.at[b], xsems[b])
        for g in range(CHUNK // LANES):
            tvec = lax.shift_right_logical(
                idx_v[c, pl.ds(g * LANES, LANES)], 3)
            for l in range(LANES):
                j = g * LANES + l
                if j % 2 == 0:
                    pltpu.async_copy(
                        table_hbm.at[tvec[l]],
                        gath_v.at[b].at[j // 2],
                        gsems[b])
                else:
                    pltpu.async_copy(
                        table_hbm.at[tvec[l]],
                        spm.at[sid].at[b].at[j // 2],
                        ssems[b])

    def process_chunk(c, b):
        pltpu.make_async_copy(
            table_hbm.at[pl.ds(0, HCH)], gath_v.at[b], gsems[b]).wait()
        pltpu.make_async_copy(
            table_hbm.at[pl.ds(0, HCH)], spm.at[sid].at[b],
            ssems[b]).wait()
        pltpu.sync_copy(spm.at[sid].at[b], gath2_v.at[b])
        pltpu.make_async_copy(
            x_hbm.at[wid].at[0], x_v.at[b], xsems[b]).wait()
        @pl.when(c >= 2)
        def _():
            pltpu.make_async_copy(
                out_v.at[b], out_hbm.at[wid].at[0], osems[b]).wait()

        for g in range(CHUNK // LANES):
            svec = lax.bitwise_and(idx_v[c, pl.ds(g * LANES, LANES)], 7)
            for l in range(LANES):
                s = svec[l]
                j = g * LANES + l
                src = gath_v if j % 2 == 0 else gath2_v
                for d in range(EMBED // LANES):
                    dsl = pl.ds(d * LANES, LANES)
                    out_v[b, j, dsl] = (
                        src[b, j // 2, s, dsl] * x_v[b, j, dsl])

        pltpu.async_copy(out_v.at[b], out_hbm.at[wid].at[c], osems[b])

    issue_chunk(0, 0)

    def pair_body(i, carry):
        c0 = i * 2
        issue_chunk(c0 + 1, 1)
        process_chunk(c0, 0)

        @pl.when(c0 + 2 < NCH)
        def _():
            issue_chunk(c0 + 2, 0)

        process_chunk(c0 + 1, 1)
        return carry

    lax.fori_loop(0, NCH // 2, pair_body, 0)

    for b in range(2):
        pltpu.make_async_copy(
            out_v.at[b], out_hbm.at[wid].at[0], osems[b]).wait()


def kernel(x, id, table):
    idx = id.astype(jnp.int32).reshape(NW, NCH, CHUNK)
    x_r = x.reshape(NW, NCH, CHUNK, EMBED)
    table_t = table.reshape(table.shape[0] // TILE_R, TILE_R, EMBED)
    out = _sc_embed(x_r, idx, table_t)
    return out.reshape(BATCH, EMBED)


# speedup vs baseline: 2.4749x; 1.0552x over previous
import functools

import jax
import jax.numpy as jnp
from jax import lax
from jax.experimental import pallas as pl
from jax.experimental.pallas import tpu as pltpu
from jax.experimental.pallas import tpu_sc as plsc

EMBED = 64
BATCH = 16384
LANES = 16
TILE_R = 2
SHIFT = 1
MASK = TILE_R - 1
NUM_CORES = 2
NUM_SUBCORES = 16
NW = NUM_CORES * NUM_SUBCORES
CHUNK = 32
NCH = BATCH // (NW * CHUNK)

_MESH = plsc.VectorSubcoreMesh(
    core_axis_name="c", subcore_axis_name="s",
    num_cores=NUM_CORES, num_subcores=NUM_SUBCORES)


@functools.partial(
    pl.kernel,
    out_type=jax.ShapeDtypeStruct((NW, NCH, CHUNK, EMBED), jnp.float32),
    mesh=_MESH,
    scratch_types=[
        pltpu.VMEM((NCH, CHUNK), jnp.int32),
        pltpu.VMEM((2, CHUNK, TILE_R, EMBED), jnp.float32),
        pltpu.VMEM((2, CHUNK, EMBED), jnp.float32),
        pltpu.VMEM((2, CHUNK, EMBED), jnp.float32),
        [pltpu.SemaphoreType.DMA] * 2,
        [pltpu.SemaphoreType.DMA] * 2,
        [pltpu.SemaphoreType.DMA] * 2,
    ],
)
def _sc_embed(x_hbm, idx_hbm, table_hbm, out_hbm,
              idx_v, gath_v, x_v, out_v, gsems, xsems, osems):
    wid = lax.axis_index("s") * NUM_CORES + lax.axis_index("c")

    pltpu.sync_copy(idx_hbm.at[wid], idx_v)

    def issue_chunk(c, b):
        pltpu.async_copy(x_hbm.at[wid].at[c], x_v.at[b], xsems[b])
        for g in range(CHUNK // LANES):
            tvec = lax.shift_right_logical(
                idx_v[c, pl.ds(g * LANES, LANES)], SHIFT)
            for l in range(LANES):
                pltpu.async_copy(
                    table_hbm.at[tvec[l]],
                    gath_v.at[b].at[g * LANES + l],
                    gsems[b])

    def process_chunk(c, b):
        pltpu.make_async_copy(
            table_hbm.at[pl.ds(0, CHUNK)], gath_v.at[b], gsems[b]).wait()
        pltpu.make_async_copy(
            x_hbm.at[wid].at[0], x_v.at[b], xsems[b]).wait()
        @pl.when(c >= 2)
        def _():
            pltpu.make_async_copy(
                out_v.at[b], out_hbm.at[wid].at[0], osems[b]).wait()

        for g in range(CHUNK // LANES):
            svec = lax.bitwise_and(
                idx_v[c, pl.ds(g * LANES, LANES)], MASK)
            for l in range(LANES):
                s = svec[l]
                j = g * LANES + l
                for d in range(EMBED // LANES):
                    dsl = pl.ds(d * LANES, LANES)
                    out_v[b, j, dsl] = gath_v[b, j, s, dsl] * x_v[b, j, dsl]

        pltpu.async_copy(out_v.at[b], out_hbm.at[wid].at[c], osems[b])

    issue_chunk(0, 0)

    def pair_body(i, carry):
        c0 = i * 2
        issue_chunk(c0 + 1, 1)
        process_chunk(c0, 0)

        @pl.when(c0 + 2 < NCH)
        def _():
            issue_chunk(c0 + 2, 0)

        process_chunk(c0 + 1, 1)
        return carry

    lax.fori_loop(0, NCH // 2, pair_body, 0)

    for b in range(2):
        pltpu.make_async_copy(
            out_v.at[b], out_hbm.at[wid].at[0], osems[b]).wait()


def kernel(x, id, table):
    idx = id.astype(jnp.int32).reshape(NW, NCH, CHUNK)
    x_r = x.reshape(NW, NCH, CHUNK, EMBED)
    table_t = table.reshape(table.shape[0] // TILE_R, TILE_R, EMBED)
    out = _sc_embed(x_r, idx, table_t)
    return out.reshape(BATCH, EMBED)


# speedup vs baseline: 2.4781x; 1.0013x over previous
import functools

import jax
import jax.numpy as jnp
from jax import lax
from jax.experimental import pallas as pl
from jax.experimental.pallas import tpu as pltpu
from jax.experimental.pallas import tpu_sc as plsc

EMBED = 64
BATCH = 16384
LANES = 16
TILE_R = 2
SHIFT = 1
MASK = TILE_R - 1
NUM_CORES = 2
NUM_SUBCORES = 16
NW = NUM_CORES * NUM_SUBCORES
CHUNK = 32
NCH = BATCH // (NW * CHUNK)

_MESH = plsc.VectorSubcoreMesh(
    core_axis_name="c", subcore_axis_name="s",
    num_cores=NUM_CORES, num_subcores=NUM_SUBCORES)


@functools.partial(
    pl.kernel,
    out_type=jax.ShapeDtypeStruct((NW, NCH, CHUNK, EMBED), jnp.float32),
    mesh=_MESH,
    scratch_types=[
        pltpu.VMEM((NCH, CHUNK), jnp.int32),
        pltpu.VMEM((2, CHUNK, TILE_R, EMBED), jnp.float32),
        pltpu.VMEM((2, CHUNK, EMBED), jnp.float32),
        pltpu.VMEM((2, CHUNK, EMBED), jnp.float32),
        [[pltpu.SemaphoreType.DMA] * 4] * 2,
        [pltpu.SemaphoreType.DMA] * 2,
        [pltpu.SemaphoreType.DMA] * 2,
    ],
)
def _sc_embed(x_hbm, idx_hbm, table_hbm, out_hbm,
              idx_v, gath_v, x_v, out_v, gsems, xsems, osems):
    wid = lax.axis_index("s") * NUM_CORES + lax.axis_index("c")

    pltpu.sync_copy(idx_hbm.at[wid], idx_v)

    def issue_chunk(c, b):
        pltpu.async_copy(x_hbm.at[wid].at[c], x_v.at[b], xsems[b])
        for g in range(CHUNK // LANES):
            tvec = lax.shift_right_logical(
                idx_v[c, pl.ds(g * LANES, LANES)], SHIFT)
            for l in range(LANES):
                pltpu.async_copy(
                    table_hbm.at[tvec[l]],
                    gath_v.at[b].at[g * LANES + l],
                    gsems[b][l % 4])

    def process_chunk(c, b):
        for q in range(4):
            pltpu.make_async_copy(
                table_hbm.at[pl.ds(0, CHUNK // 4)],
                gath_v.at[b].at[pl.ds(0, CHUNK // 4)], gsems[b][q]).wait()
        pltpu.make_async_copy(
            x_hbm.at[wid].at[0], x_v.at[b], xsems[b]).wait()
        @pl.when(c >= 2)
        def _():
            pltpu.make_async_copy(
                out_v.at[b], out_hbm.at[wid].at[0], osems[b]).wait()

        for g in range(CHUNK // LANES):
            svec = lax.bitwise_and(
                idx_v[c, pl.ds(g * LANES, LANES)], MASK)
            for l in range(LANES):
                s = svec[l]
                j = g * LANES + l
                for d in range(EMBED // LANES):
                    dsl = pl.ds(d * LANES, LANES)
                    out_v[b, j, dsl] = gath_v[b, j, s, dsl] * x_v[b, j, dsl]

        pltpu.async_copy(out_v.at[b], out_hbm.at[wid].at[c], osems[b])

    issue_chunk(0, 0)

    def pair_body(i, carry):
        c0 = i * 2
        issue_chunk(c0 + 1, 1)
        process_chunk(c0, 0)

        @pl.when(c0 + 2 < NCH)
        def _():
            issue_chunk(c0 + 2, 0)

        process_chunk(c0 + 1, 1)
        return carry

    lax.fori_loop(0, NCH // 2, pair_body, 0)

    for b in range(2):
        pltpu.make_async_copy(
            out_v.at[b], out_hbm.at[wid].at[0], osems[b]).wait()


def kernel(x, id, table):
    idx = id.astype(jnp.int32).reshape(NW, NCH, CHUNK)
    x_r = x.reshape(NW, NCH, CHUNK, EMBED)
    table_t = table.reshape(table.shape[0] // TILE_R, TILE_R, EMBED)
    out = _sc_embed(x_r, idx, table_t)
    return out.reshape(BATCH, EMBED)
